# speedup vs baseline: 17.0811x; 17.0811x over previous
import functools

import jax
import jax.numpy as jnp
from jax import lax
from jax.experimental import pallas as pl
from jax.experimental.pallas import tpu as pltpu
from jax.experimental.pallas import tpu_sc as plsc

_D = 128
_C = 40
_CK = 128
_NSUB = 16
_NCORE = 2
_NW = _NCORE * _NSUB
_LANES = 16
_DEGW = 16


def _sc_degree(dst3, npad):
    nw, ch, ck = dst3.shape
    rpt = npad // _NSUB
    mesh = plsc.VectorSubcoreMesh(core_axis_name="c", subcore_axis_name="s")

    @functools.partial(
        pl.kernel,
        out_type=jax.ShapeDtypeStruct((2 * npad, _DEGW), jnp.float32),
        mesh=mesh,
        scratch_types=[
            pltpu.VMEM((ch, ck), jnp.int32),
            pltpu.VMEM((_CK, _DEGW), jnp.float32),
            pltpu.VMEM_SHARED((npad, _DEGW), jnp.float32),
            pltpu.SemaphoreType.DMA,
        ],
    )
    def k(dst_hbm, out_hbm, dst_v, buf_v, acc_sh, sem):
        cid = lax.axis_index("c")
        sid = lax.axis_index("s")
        wid = cid * _NSUB + sid

        @pl.loop(0, _CK)
        def _(i):
            buf_v.at[i][...] = jnp.zeros((_DEGW,), jnp.float32)

        @pl.loop(0, rpt // _CK)
        def _(j):
            pltpu.sync_copy(buf_v, acc_sh.at[pl.ds(sid * rpt + j * _CK, _CK)])

        @pl.loop(0, _CK)
        def _(i):
            buf_v.at[i][...] = jnp.ones((_DEGW,), jnp.float32)

        pltpu.sync_copy(dst_hbm.at[wid], dst_v)
        plsc.subcore_barrier()

        @pl.loop(0, ch)
        def _(j):
            pltpu.sync_copy(buf_v, acc_sh.at[dst_v.at[j]], add=True)

        plsc.subcore_barrier()

        @pl.loop(0, rpt // _CK)
        def _(j):
            r0 = sid * rpt + j * _CK
            pltpu.sync_copy(acc_sh.at[pl.ds(r0, _CK)], buf_v)
            pltpu.sync_copy(buf_v, out_hbm.at[pl.ds(cid * npad + r0, _CK)])

    return k(dst3)


def _sc_propagate(hs, src3, dst3):
    npad = hs.shape[0]
    nw, ch, ck = src3.shape
    rpt = npad // _NSUB
    mesh = plsc.VectorSubcoreMesh(core_axis_name="c", subcore_axis_name="s")

    @functools.partial(
        pl.kernel,
        out_type=jax.ShapeDtypeStruct((2 * npad, _D), jnp.float32),
        mesh=mesh,
        scratch_types=[
            pltpu.VMEM((ch, ck), jnp.int32),
            pltpu.VMEM((ch, ck), jnp.int32),
            pltpu.VMEM((_CK, _D), jnp.float32),
            pltpu.VMEM_SHARED((npad, _D), jnp.float32),
            pltpu.SemaphoreType.DMA,
        ],
    )
    def k(hs_hbm, src_hbm, dst_hbm, out_hbm, src_v, dst_v, rows_v, acc_sh, sem):
        cid = lax.axis_index("c")
        sid = lax.axis_index("s")
        wid = cid * _NSUB + sid

        @pl.loop(0, _CK)
        def _(i):
            @pl.loop(0, _D // _LANES)
            def _(j):
                rows_v.at[i, pl.ds(j * _LANES, _LANES)][...] = jnp.zeros(
                    (_LANES,), jnp.float32)

        @pl.loop(0, rpt // _CK)
        def _(j):
            pltpu.sync_copy(rows_v, acc_sh.at[pl.ds(sid * rpt + j * _CK, _CK)])

        pltpu.sync_copy(src_hbm.at[wid], src_v)
        pltpu.sync_copy(dst_hbm.at[wid], dst_v)
        plsc.subcore_barrier()

        @pl.loop(0, ch)
        def _(j):
            pltpu.async_copy(hs_hbm.at[src_v.at[j]], rows_v, sem).wait()
            pltpu.sync_copy(rows_v, acc_sh.at[dst_v.at[j]], add=True)

        plsc.subcore_barrier()

        @pl.loop(0, rpt // _CK)
        def _(j):
            r0 = sid * rpt + j * _CK
            pltpu.sync_copy(acc_sh.at[pl.ds(r0, _CK)], rows_v)
            pltpu.sync_copy(rows_v, out_hbm.at[pl.ds(cid * npad + r0, _CK)])

    return k(hs, src3, dst3)


_HI = lax.Precision.HIGHEST


def _prep_body(d0_ref, d1_ref, x_ref, w_ref, b_ref, dis_ref, xs_ref, lg_ref):
    deg = d0_ref[:, 0:1] + d1_ref[:, 0:1] + 1.0
    dis = lax.rsqrt(deg)
    dis_ref[...] = dis
    xs_ref[...] = x_ref[...] * dis
    lg_ref[...] = jnp.dot(x_ref[...], w_ref[...],
                          preferred_element_type=jnp.float32,
                          precision=_HI) + b_ref[...]


def _tc_prep(degp, x_pad, w0, bout, npad, rblk):
    nb = npad // rblk
    return pl.pallas_call(
        _prep_body,
        grid=(nb,),
        in_specs=[
            pl.BlockSpec((rblk, _DEGW), lambda i: (i, 0)),
            pl.BlockSpec((rblk, _DEGW), lambda i, nb=nb: (i + nb, 0)),
            pl.BlockSpec((rblk, _D), lambda i: (i, 0)),
            pl.BlockSpec((_D, _C), lambda i: (0, 0)),
            pl.BlockSpec((1, _C), lambda i: (0, 0)),
        ],
        out_specs=[
            pl.BlockSpec((rblk, 1), lambda i: (i, 0)),
            pl.BlockSpec((rblk, _D), lambda i: (i, 0)),
            pl.BlockSpec((rblk, _C), lambda i: (i, 0)),
        ],
        out_shape=[
            jax.ShapeDtypeStruct((npad, 1), jnp.float32),
            jax.ShapeDtypeStruct((npad, _D), jnp.float32),
            jax.ShapeDtypeStruct((npad, _C), jnp.float32),
        ],
    )(degp, degp, x_pad, w0, bout)


def _layer_body(p0_ref, p1_ref, hs_ref, dis_ref, w_ref, b_ref, wo_ref,
                lgin_ref, hso_ref, lgo_ref):
    dis = dis_ref[...]
    agg = (p0_ref[...] + p1_ref[...] + hs_ref[...]) * dis
    h = jnp.maximum(
        jnp.dot(agg, w_ref[...], preferred_element_type=jnp.float32,
                precision=_HI) + b_ref[...], 0.0)
    hso_ref[...] = h * dis
    lgo_ref[...] = lgin_ref[...] + jnp.dot(
        h, wo_ref[...], preferred_element_type=jnp.float32, precision=_HI)


def _tc_layer(parts, hs, dis, w, b, wo, lgin, npad, rblk):
    nb = npad // rblk
    return pl.pallas_call(
        _layer_body,
        grid=(nb,),
        in_specs=[
            pl.BlockSpec((rblk, _D), lambda i: (i, 0)),
            pl.BlockSpec((rblk, _D), lambda i, nb=nb: (i + nb, 0)),
            pl.BlockSpec((rblk, _D), lambda i: (i, 0)),
            pl.BlockSpec((rblk, 1), lambda i: (i, 0)),
            pl.BlockSpec((_D, _D), lambda i: (0, 0)),
            pl.BlockSpec((1, _D), lambda i: (0, 0)),
            pl.BlockSpec((_D, _C), lambda i: (0, 0)),
            pl.BlockSpec((rblk, _C), lambda i: (i, 0)),
        ],
        out_specs=[
            pl.BlockSpec((rblk, _D), lambda i: (i, 0)),
            pl.BlockSpec((rblk, _C), lambda i: (i, 0)),
        ],
        out_shape=[
            jax.ShapeDtypeStruct((npad, _D), jnp.float32),
            jax.ShapeDtypeStruct((npad, _C), jnp.float32),
        ],
    )(parts, parts, hs, dis, w, b, wo, lgin)


def kernel(x, edge_index, W1, b1, W2, b2, W3, b3, Wout, bout):
    n, d = x.shape
    e = edge_index.shape[1]
    npad = ((n + 16 * _CK - 1) // (16 * _CK)) * (16 * _CK)
    epw = ((e + _NW * _CK - 1) // (_NW * _CK)) * _CK
    epad = epw * _NW

    pad = epad - e
    pad_idx = n + (jnp.arange(pad, dtype=jnp.int32) % (npad - n))
    src_p = jnp.concatenate([edge_index[0], pad_idx]).reshape(_NW, epw // _CK, _CK)
    dst_p = jnp.concatenate([edge_index[1], pad_idx]).reshape(_NW, epw // _CK, _CK)

    x_pad = jnp.pad(x, ((0, npad - n), (0, 0)))
    bout2 = bout.reshape(1, _C)

    rblk = 1024

    degp = _sc_degree(dst_p, npad)
    dis, xs, lg0 = _tc_prep(degp, x_pad, Wout[0:_D], bout2, npad, rblk)

    p1 = _sc_propagate(xs, src_p, dst_p)
    hs1, lg1 = _tc_layer(p1, xs, dis, W1, b1.reshape(1, _D),
                         Wout[_D:2 * _D], lg0, npad, rblk)

    p2 = _sc_propagate(hs1, src_p, dst_p)
    hs2, lg2 = _tc_layer(p2, hs1, dis, W2, b2.reshape(1, _D),
                         Wout[2 * _D:3 * _D], lg1, npad, rblk)

    p3 = _sc_propagate(hs2, src_p, dst_p)
    _, lg3 = _tc_layer(p3, hs2, dis, W3, b3.reshape(1, _D),
                       Wout[3 * _D:4 * _D], lg2, npad, rblk)

    return lg3[:n]


# speedup vs baseline: 21.3006x; 1.2470x over previous
import functools

import jax
import jax.numpy as jnp
from jax import lax
from jax.experimental import pallas as pl
from jax.experimental.pallas import tpu as pltpu
from jax.experimental.pallas import tpu_sc as plsc

_D = 128
_C = 40
_CK = 128
_NSUB = 16
_NCORE = 2
_NW = _NCORE * _NSUB
_LANES = 16
_DEGW = 16


def _sc_degree(dst3, npad):
    nwch, ck = dst3.shape
    ch = nwch // _NW
    rpt = npad // _NSUB
    mesh = plsc.VectorSubcoreMesh(core_axis_name="c", subcore_axis_name="s")

    @functools.partial(
        pl.kernel,
        out_type=jax.ShapeDtypeStruct((2 * npad, _DEGW), jnp.float32),
        mesh=mesh,
        scratch_types=[
            pltpu.VMEM((ch, ck), jnp.int32),
            pltpu.VMEM((_CK, _DEGW), jnp.float32),
            pltpu.VMEM_SHARED((npad, _DEGW), jnp.float32),
            pltpu.SemaphoreType.DMA,
        ],
    )
    def k(dst_hbm, out_hbm, dst_v, buf_v, acc_sh, sem):
        cid = lax.axis_index("c")
        sid = lax.axis_index("s")
        wid = cid * _NSUB + sid

        @pl.loop(0, _CK)
        def _(i):
            buf_v.at[i][...] = jnp.zeros((_DEGW,), jnp.float32)

        @pl.loop(0, rpt // _CK)
        def _(j):
            pltpu.sync_copy(buf_v, acc_sh.at[pl.ds(sid * rpt + j * _CK, _CK)])

        @pl.loop(0, _CK)
        def _(i):
            buf_v.at[i][...] = jnp.ones((_DEGW,), jnp.float32)

        pltpu.sync_copy(dst_hbm.at[pl.ds(wid * ch, ch)], dst_v)
        plsc.subcore_barrier()

        @pl.loop(0, ch)
        def _(j):
            pltpu.sync_copy(buf_v, acc_sh.at[dst_v.at[j]], add=True)

        plsc.subcore_barrier()

        @pl.loop(0, rpt // _CK)
        def _(j):
            r0 = sid * rpt + j * _CK
            pltpu.sync_copy(acc_sh.at[pl.ds(r0, _CK)], buf_v)
            pltpu.sync_copy(buf_v, out_hbm.at[pl.ds(cid * npad + r0, _CK)])

    return k(dst3)


def _sc_propagate(hs, src3, dst3):
    npad = hs.shape[0]
    nwch, ck = src3.shape
    ch = nwch // _NW
    rpt = npad // _NSUB
    mesh = plsc.VectorSubcoreMesh(core_axis_name="c", subcore_axis_name="s")

    nph = 2
    chp = ch // nph
    assert ch % (nph * 2) == 0

    @functools.partial(
        pl.kernel,
        out_type=jax.ShapeDtypeStruct((2 * npad, _D), jnp.float32),
        mesh=mesh,
        scratch_types=[
            pltpu.VMEM((chp, ck), jnp.int32),
            pltpu.VMEM((chp, ck), jnp.int32),
            pltpu.VMEM_SHARED((npad, _D), jnp.float32),
            pltpu.VMEM((_CK, _D), jnp.float32),
            pltpu.VMEM((_CK, _D), jnp.float32),
        ] + [pltpu.SemaphoreType.DMA] * 4,
    )
    def k(hs_hbm, src_hbm, dst_hbm, out_hbm, src_v, dst_v, acc_sh,
          rows0, rows1, gsem0, gsem1, ssem0, ssem1):
        rows = (rows0, rows1)
        gsem = (gsem0, gsem1)
        ssem = (ssem0, ssem1)
        cid = lax.axis_index("c")
        sid = lax.axis_index("s")
        wid = cid * _NSUB + sid

        @pl.loop(0, _CK)
        def _(i):
            @pl.loop(0, _D // _LANES)
            def _(j):
                rows0.at[i, pl.ds(j * _LANES, _LANES)][...] = jnp.zeros(
                    (_LANES,), jnp.float32)

        @pl.loop(0, rpt // _CK)
        def _(j):
            pltpu.sync_copy(rows0,
                            acc_sh.at[pl.ds(sid * rpt + j * _CK, _CK)])

        plsc.subcore_barrier()

        def g_start(w, b):
            pltpu.async_copy(hs_hbm.at[src_v.at[w]], rows[b], gsem[b])

        def g_wait(w, b):
            pltpu.make_async_copy(hs_hbm.at[src_v.at[w]], rows[b],
                                  gsem[b]).wait()

        def s_start(w, b):
            return pltpu.async_copy(rows[b], acc_sh.at[dst_v.at[w]], ssem[b],
                                    add=True)

        def s_wait(w, b):
            pltpu.make_async_copy(rows[b], acc_sh.at[dst_v.at[w]],
                                  ssem[b]).wait()

        for p in range(nph):
            base = wid * ch + p * chp
            pltpu.sync_copy(src_hbm.at[pl.ds(base, chp)], src_v)
            pltpu.sync_copy(dst_hbm.at[pl.ds(base, chp)], dst_v)

            g_start(0, 0)

            @pl.loop(0, chp, step=2)
            def _(j):
                for b in range(2):
                    w = j + b
                    g_wait(w, b)
                    cp = s_start(w, b)

                    @pl.when(w + 1 < chp)
                    def _():
                        g_start(w + 1, 1 - b)

                    cp.wait()

        plsc.subcore_barrier()

        @pl.loop(0, rpt // _CK)
        def _(j):
            r0 = sid * rpt + j * _CK
            pltpu.sync_copy(acc_sh.at[pl.ds(r0, _CK)], rows0)
            pltpu.sync_copy(rows0, out_hbm.at[pl.ds(cid * npad + r0, _CK)])

    return k(hs, src3, dst3)


_HI = lax.Precision.HIGHEST


def _prep_body(d0_ref, d1_ref, x_ref, w_ref, b_ref, dis_ref, xs_ref, lg_ref):
    deg = d0_ref[:, 0:1] + d1_ref[:, 0:1] + 1.0
    dis = lax.rsqrt(deg)
    dis_ref[...] = dis
    xs_ref[...] = x_ref[...] * dis
    lg_ref[...] = jnp.dot(x_ref[...], w_ref[...],
                          preferred_element_type=jnp.float32,
                          precision=_HI) + b_ref[...]


def _tc_prep(degp, x_pad, w0, bout, npad, rblk):
    nb = npad // rblk
    return pl.pallas_call(
        _prep_body,
        grid=(nb,),
        in_specs=[
            pl.BlockSpec((rblk, _DEGW), lambda i: (i, 0)),
            pl.BlockSpec((rblk, _DEGW), lambda i, nb=nb: (i + nb, 0)),
            pl.BlockSpec((rblk, _D), lambda i: (i, 0)),
            pl.BlockSpec((_D, _C), lambda i: (0, 0)),
            pl.BlockSpec((1, _C), lambda i: (0, 0)),
        ],
        out_specs=[
            pl.BlockSpec((rblk, 1), lambda i: (i, 0)),
            pl.BlockSpec((rblk, _D), lambda i: (i, 0)),
            pl.BlockSpec((rblk, _C), lambda i: (i, 0)),
        ],
        out_shape=[
            jax.ShapeDtypeStruct((npad, 1), jnp.float32),
            jax.ShapeDtypeStruct((npad, _D), jnp.float32),
            jax.ShapeDtypeStruct((npad, _C), jnp.float32),
        ],
    )(degp, degp, x_pad, w0, bout)


def _layer_body(p0_ref, p1_ref, hs_ref, dis_ref, w_ref, b_ref, wo_ref,
                lgin_ref, hso_ref, lgo_ref):
    dis = dis_ref[...]
    agg = (p0_ref[...] + p1_ref[...] + hs_ref[...]) * dis
    h = jnp.maximum(
        jnp.dot(agg, w_ref[...], preferred_element_type=jnp.float32,
                precision=_HI) + b_ref[...], 0.0)
    hso_ref[...] = h * dis
    lgo_ref[...] = lgin_ref[...] + jnp.dot(
        h, wo_ref[...], preferred_element_type=jnp.float32, precision=_HI)


def _tc_layer(parts, hs, dis, w, b, wo, lgin, npad, rblk):
    nb = npad // rblk
    return pl.pallas_call(
        _layer_body,
        grid=(nb,),
        in_specs=[
            pl.BlockSpec((rblk, _D), lambda i: (i, 0)),
            pl.BlockSpec((rblk, _D), lambda i, nb=nb: (i + nb, 0)),
            pl.BlockSpec((rblk, _D), lambda i: (i, 0)),
            pl.BlockSpec((rblk, 1), lambda i: (i, 0)),
            pl.BlockSpec((_D, _D), lambda i: (0, 0)),
            pl.BlockSpec((1, _D), lambda i: (0, 0)),
            pl.BlockSpec((_D, _C), lambda i: (0, 0)),
            pl.BlockSpec((rblk, _C), lambda i: (i, 0)),
        ],
        out_specs=[
            pl.BlockSpec((rblk, _D), lambda i: (i, 0)),
            pl.BlockSpec((rblk, _C), lambda i: (i, 0)),
        ],
        out_shape=[
            jax.ShapeDtypeStruct((npad, _D), jnp.float32),
            jax.ShapeDtypeStruct((npad, _C), jnp.float32),
        ],
    )(parts, parts, hs, dis, w, b, wo, lgin)


def kernel(x, edge_index, W1, b1, W2, b2, W3, b3, Wout, bout):
    n, d = x.shape
    e = edge_index.shape[1]
    npad = ((n + 16 * _CK - 1) // (16 * _CK)) * (16 * _CK)
    gran = _NW * _CK * 4
    epw = ((e + gran - 1) // gran) * gran // _NW
    epad = epw * _NW

    pad = epad - e
    pad_idx = n + (jnp.arange(pad, dtype=jnp.int32) % (npad - n))
    src_p = jnp.concatenate([edge_index[0], pad_idx]).reshape(epad // _CK, _CK)
    dst_p = jnp.concatenate([edge_index[1], pad_idx]).reshape(epad // _CK, _CK)

    x_pad = jnp.pad(x, ((0, npad - n), (0, 0)))
    bout2 = bout.reshape(1, _C)

    rblk = 1024

    degp = _sc_degree(dst_p, npad)
    dis, xs, lg0 = _tc_prep(degp, x_pad, Wout[0:_D], bout2, npad, rblk)

    p1 = _sc_propagate(xs, src_p, dst_p)
    hs1, lg1 = _tc_layer(p1, xs, dis, W1, b1.reshape(1, _D),
                         Wout[_D:2 * _D], lg0, npad, rblk)

    p2 = _sc_propagate(hs1, src_p, dst_p)
    hs2, lg2 = _tc_layer(p2, hs1, dis, W2, b2.reshape(1, _D),
                         Wout[2 * _D:3 * _D], lg1, npad, rblk)

    p3 = _sc_propagate(hs2, src_p, dst_p)
    _, lg3 = _tc_layer(p3, hs2, dis, W3, b3.reshape(1, _D),
                       Wout[3 * _D:4 * _D], lg2, npad, rblk)

    return lg3[:n]


# speedup vs baseline: 22.9960x; 1.0796x over previous
import functools

import jax
import jax.numpy as jnp
from jax import lax
from jax.experimental import pallas as pl
from jax.experimental.pallas import tpu as pltpu
from jax.experimental.pallas import tpu_sc as plsc

_D = 128
_C = 40
_CK = 128
_NSUB = 16
_NCORE = 2
_NW = _NCORE * _NSUB
_LANES = 16
_DEGW = 16


def _sc_degree(dst3, npad):
    nwch, ck = dst3.shape
    ch = nwch // _NW
    rpt = npad // _NSUB
    mesh = plsc.VectorSubcoreMesh(core_axis_name="c", subcore_axis_name="s")

    @functools.partial(
        pl.kernel,
        out_type=jax.ShapeDtypeStruct((2 * npad, _DEGW), jnp.float32),
        mesh=mesh,
        scratch_types=[
            pltpu.VMEM((ch, ck), jnp.int32),
            pltpu.VMEM((ck, _DEGW), jnp.float32),
            pltpu.VMEM_SHARED((npad, _DEGW), jnp.float32),
            pltpu.SemaphoreType.DMA,
        ],
    )
    def k(dst_hbm, out_hbm, dst_v, buf_v, acc_sh, sem):
        cid = lax.axis_index("c")
        sid = lax.axis_index("s")
        wid = cid * _NSUB + sid

        @pl.loop(0, ck)
        def _(i):
            buf_v.at[i][...] = jnp.zeros((_DEGW,), jnp.float32)

        @pl.loop(0, rpt // ck)
        def _(j):
            pltpu.sync_copy(buf_v, acc_sh.at[pl.ds(sid * rpt + j * ck, ck)])

        @pl.loop(0, ck)
        def _(i):
            buf_v.at[i][...] = jnp.ones((_DEGW,), jnp.float32)

        pltpu.sync_copy(dst_hbm.at[pl.ds(wid * ch, ch)], dst_v)
        plsc.subcore_barrier()

        @pl.loop(0, ch)
        def _(j):
            pltpu.sync_copy(buf_v, acc_sh.at[dst_v.at[j]], add=True)

        plsc.subcore_barrier()

        @pl.loop(0, rpt // ck)
        def _(j):
            r0 = sid * rpt + j * ck
            pltpu.sync_copy(acc_sh.at[pl.ds(r0, ck)], buf_v)
            pltpu.sync_copy(buf_v, out_hbm.at[pl.ds(cid * npad + r0, ck)])

    return k(dst3)


def _sc_propagate(hs, src3, dst3):
    npad = hs.shape[0]
    nwch, ck = src3.shape
    ch = nwch // _NW
    rpt = npad // _NSUB
    mesh = plsc.VectorSubcoreMesh(core_axis_name="c", subcore_axis_name="s")

    nslot = 4
    nph = 4
    chp = ch // nph
    assert ch % (nph * nslot) == 0

    @functools.partial(
        pl.kernel,
        out_type=jax.ShapeDtypeStruct((2 * npad, _D), jnp.float32),
        mesh=mesh,
        scratch_types=[
            pltpu.VMEM((chp, ck), jnp.int32),
            pltpu.VMEM((chp, ck), jnp.int32),
            pltpu.VMEM_SHARED((npad, _D), jnp.float32),
        ] + [pltpu.VMEM((ck, _D), jnp.float32)] * nslot
          + [pltpu.SemaphoreType.DMA] * (2 * nslot),
    )
    def k(hs_hbm, src_hbm, dst_hbm, out_hbm, src_v, dst_v, acc_sh, *rest):
        rows = rest[:nslot]
        gsem = rest[nslot:2 * nslot]
        ssem = rest[2 * nslot:]
        rows0 = rows[0]
        cid = lax.axis_index("c")
        sid = lax.axis_index("s")
        wid = cid * _NSUB + sid

        @pl.loop(0, ck)
        def _(i):
            @pl.loop(0, _D // _LANES)
            def _(j):
                rows0.at[i, pl.ds(j * _LANES, _LANES)][...] = jnp.zeros(
                    (_LANES,), jnp.float32)

        @pl.loop(0, rpt // ck)
        def _(j):
            pltpu.sync_copy(rows0,
                            acc_sh.at[pl.ds(sid * rpt + j * ck, ck)])

        plsc.subcore_barrier()

        def g_start(w, b):
            pltpu.async_copy(hs_hbm.at[src_v.at[w]], rows[b], gsem[b])

        def g_wait(w, b):
            pltpu.make_async_copy(hs_hbm.at[src_v.at[w]], rows[b],
                                  gsem[b]).wait()

        def s_start(w, b):
            return pltpu.async_copy(rows[b], acc_sh.at[dst_v.at[w]], ssem[b],
                                    add=True)

        def s_wait(w, b):
            pltpu.make_async_copy(rows[b], acc_sh.at[dst_v.at[w]],
                                  ssem[b]).wait()

        for p in range(nph):
            base = wid * ch + p * chp
            pltpu.sync_copy(src_hbm.at[pl.ds(base, chp)], src_v)
            pltpu.sync_copy(dst_hbm.at[pl.ds(base, chp)], dst_v)

            g_start(0, 0)
            g_start(1, 1)

            @pl.loop(0, chp, step=nslot)
            def _(j):
                for b in range(nslot):
                    w = j + b
                    g_wait(w, b)
                    cp = s_start(w, b)

                    @pl.when(w + 2 < chp)
                    def _():
                        g_start(w + 2, (b + 2) % nslot)

                    cp.wait()

        plsc.subcore_barrier()

        @pl.loop(0, rpt // ck)
        def _(j):
            r0 = sid * rpt + j * ck
            pltpu.sync_copy(acc_sh.at[pl.ds(r0, ck)], rows0)
            pltpu.sync_copy(rows0, out_hbm.at[pl.ds(cid * npad + r0, ck)])

    return k(hs, src3, dst3)


_HI = lax.Precision.HIGHEST


def _prep_body(d0_ref, d1_ref, x_ref, w_ref, b_ref, dis_ref, xs_ref, lg_ref):
    deg = d0_ref[:, 0:1] + d1_ref[:, 0:1] + 1.0
    dis = lax.rsqrt(deg)
    dis_ref[...] = dis
    xs_ref[...] = x_ref[...] * dis
    lg_ref[...] = jnp.dot(x_ref[...], w_ref[...],
                          preferred_element_type=jnp.float32,
                          precision=_HI) + b_ref[...]


def _tc_prep(degp, x_pad, w0, bout, npad, rblk):
    nb = npad // rblk
    return pl.pallas_call(
        _prep_body,
        grid=(nb,),
        in_specs=[
            pl.BlockSpec((rblk, _DEGW), lambda i: (i, 0)),
            pl.BlockSpec((rblk, _DEGW), lambda i, nb=nb: (i + nb, 0)),
            pl.BlockSpec((rblk, _D), lambda i: (i, 0)),
            pl.BlockSpec((_D, _C), lambda i: (0, 0)),
            pl.BlockSpec((1, _C), lambda i: (0, 0)),
        ],
        out_specs=[
            pl.BlockSpec((rblk, 1), lambda i: (i, 0)),
            pl.BlockSpec((rblk, _D), lambda i: (i, 0)),
            pl.BlockSpec((rblk, _C), lambda i: (i, 0)),
        ],
        out_shape=[
            jax.ShapeDtypeStruct((npad, 1), jnp.float32),
            jax.ShapeDtypeStruct((npad, _D), jnp.float32),
            jax.ShapeDtypeStruct((npad, _C), jnp.float32),
        ],
    )(degp, degp, x_pad, w0, bout)


def _layer_body(p0_ref, p1_ref, hs_ref, dis_ref, w_ref, b_ref, wo_ref,
                lgin_ref, hso_ref, lgo_ref):
    dis = dis_ref[...]
    agg = (p0_ref[...] + p1_ref[...] + hs_ref[...]) * dis
    h = jnp.maximum(
        jnp.dot(agg, w_ref[...], preferred_element_type=jnp.float32,
                precision=_HI) + b_ref[...], 0.0)
    hso_ref[...] = h * dis
    lgo_ref[...] = lgin_ref[...] + jnp.dot(
        h, wo_ref[...], preferred_element_type=jnp.float32, precision=_HI)


def _tc_layer(parts, hs, dis, w, b, wo, lgin, npad, rblk):
    nb = npad // rblk
    return pl.pallas_call(
        _layer_body,
        grid=(nb,),
        in_specs=[
            pl.BlockSpec((rblk, _D), lambda i: (i, 0)),
            pl.BlockSpec((rblk, _D), lambda i, nb=nb: (i + nb, 0)),
            pl.BlockSpec((rblk, _D), lambda i: (i, 0)),
            pl.BlockSpec((rblk, 1), lambda i: (i, 0)),
            pl.BlockSpec((_D, _D), lambda i: (0, 0)),
            pl.BlockSpec((1, _D), lambda i: (0, 0)),
            pl.BlockSpec((_D, _C), lambda i: (0, 0)),
            pl.BlockSpec((rblk, _C), lambda i: (i, 0)),
        ],
        out_specs=[
            pl.BlockSpec((rblk, _D), lambda i: (i, 0)),
            pl.BlockSpec((rblk, _C), lambda i: (i, 0)),
        ],
        out_shape=[
            jax.ShapeDtypeStruct((npad, _D), jnp.float32),
            jax.ShapeDtypeStruct((npad, _C), jnp.float32),
        ],
    )(parts, parts, hs, dis, w, b, wo, lgin)


def kernel(x, edge_index, W1, b1, W2, b2, W3, b3, Wout, bout):
    n, d = x.shape
    e = edge_index.shape[1]
    npad = ((n + 16 * _CK - 1) // (16 * _CK)) * (16 * _CK)
    ckw = 64
    gran = _NW * ckw * 8
    epw = ((e + gran - 1) // gran) * gran // _NW
    epad = epw * _NW

    pad = epad - e
    pad_idx = n + (jnp.arange(pad, dtype=jnp.int32) % (npad - n))
    src_p = jnp.concatenate([edge_index[0], pad_idx]).reshape(epad // ckw, ckw)
    dst_p = jnp.concatenate([edge_index[1], pad_idx]).reshape(epad // ckw, ckw)

    x_pad = jnp.pad(x, ((0, npad - n), (0, 0)))
    bout2 = bout.reshape(1, _C)

    rblk = 1024

    degp = _sc_degree(dst_p, npad)
    dis, xs, lg0 = _tc_prep(degp, x_pad, Wout[0:_D], bout2, npad, rblk)

    p1 = _sc_propagate(xs, src_p, dst_p)
    hs1, lg1 = _tc_layer(p1, xs, dis, W1, b1.reshape(1, _D),
                         Wout[_D:2 * _D], lg0, npad, rblk)

    p2 = _sc_propagate(hs1, src_p, dst_p)
    hs2, lg2 = _tc_layer(p2, hs1, dis, W2, b2.reshape(1, _D),
                         Wout[2 * _D:3 * _D], lg1, npad, rblk)

    p3 = _sc_propagate(hs2, src_p, dst_p)
    _, lg3 = _tc_layer(p3, hs2, dis, W3, b3.reshape(1, _D),
                       Wout[3 * _D:4 * _D], lg2, npad, rblk)

    return lg3[:n]


# speedup vs baseline: 23.4452x; 1.0195x over previous
import functools

import jax
import jax.numpy as jnp
from jax import lax
from jax.experimental import pallas as pl
from jax.experimental.pallas import tpu as pltpu
from jax.experimental.pallas import tpu_sc as plsc

_D = 128
_C = 40
_CK = 128
_NSUB = 16
_NCORE = 2
_NW = _NCORE * _NSUB
_LANES = 16
_DEGW = 16


def _sc_degree(dst3, npad):
    nwch, ck = dst3.shape
    ch = nwch // _NW
    rpt = npad // _NSUB
    mesh = plsc.VectorSubcoreMesh(core_axis_name="c", subcore_axis_name="s")

    @functools.partial(
        pl.kernel,
        out_type=jax.ShapeDtypeStruct((2 * npad, _DEGW), jnp.float32),
        mesh=mesh,
        scratch_types=[
            pltpu.VMEM((ch, ck), jnp.int32),
            pltpu.VMEM((ck, _DEGW), jnp.float32),
            pltpu.VMEM_SHARED((npad, _DEGW), jnp.float32),
            pltpu.SemaphoreType.DMA,
        ],
    )
    def k(dst_hbm, out_hbm, dst_v, buf_v, acc_sh, sem):
        cid = lax.axis_index("c")
        sid = lax.axis_index("s")
        wid = cid * _NSUB + sid

        @pl.loop(0, ck)
        def _(i):
            buf_v.at[i][...] = jnp.zeros((_DEGW,), jnp.float32)

        @pl.loop(0, rpt // ck)
        def _(j):
            pltpu.sync_copy(buf_v, acc_sh.at[pl.ds(sid * rpt + j * ck, ck)])

        @pl.loop(0, ck)
        def _(i):
            buf_v.at[i][...] = jnp.ones((_DEGW,), jnp.float32)

        pltpu.sync_copy(dst_hbm.at[pl.ds(wid * ch, ch)], dst_v)
        plsc.subcore_barrier()

        @pl.loop(0, ch)
        def _(j):
            pltpu.async_copy(buf_v, acc_sh.at[dst_v.at[j]], sem, add=True)

        @pl.loop(0, ch)
        def _(j):
            pltpu.make_async_copy(buf_v, acc_sh.at[dst_v.at[j]], sem).wait()

        plsc.subcore_barrier()

        @pl.loop(0, rpt // ck)
        def _(j):
            r0 = sid * rpt + j * ck
            pltpu.sync_copy(acc_sh.at[pl.ds(r0, ck)], buf_v)
            pltpu.sync_copy(buf_v, out_hbm.at[pl.ds(cid * npad + r0, ck)])

    return k(dst3)


def _sc_propagate(hs, src3, dst3):
    npad = hs.shape[0]
    nwch, ck = src3.shape
    ch = nwch // _NW
    rpt = npad // _NSUB
    mesh = plsc.VectorSubcoreMesh(core_axis_name="c", subcore_axis_name="s")

    nslot = 4
    nph = 4
    chp = ch // nph
    assert ch % (nph * nslot) == 0

    @functools.partial(
        pl.kernel,
        out_type=jax.ShapeDtypeStruct((2 * npad, _D), jnp.float32),
        mesh=mesh,
        scratch_types=[
            pltpu.VMEM((chp, ck), jnp.int32),
            pltpu.VMEM((chp, ck), jnp.int32),
            pltpu.VMEM_SHARED((npad, _D), jnp.float32),
        ] + [pltpu.VMEM((ck, _D), jnp.float32)] * nslot
          + [pltpu.SemaphoreType.DMA] * (2 * nslot),
    )
    def k(hs_hbm, src_hbm, dst_hbm, out_hbm, src_v, dst_v, acc_sh, *rest):
        rows = rest[:nslot]
        gsem = rest[nslot:2 * nslot]
        ssem = rest[2 * nslot:]
        rows0 = rows[0]
        cid = lax.axis_index("c")
        sid = lax.axis_index("s")
        wid = cid * _NSUB + sid

        @pl.loop(0, ck)
        def _(i):
            @pl.loop(0, _D // _LANES)
            def _(j):
                rows0.at[i, pl.ds(j * _LANES, _LANES)][...] = jnp.zeros(
                    (_LANES,), jnp.float32)

        @pl.loop(0, rpt // ck)
        def _(j):
            pltpu.sync_copy(rows0,
                            acc_sh.at[pl.ds(sid * rpt + j * ck, ck)])

        plsc.subcore_barrier()

        def g_start(w, b):
            pltpu.async_copy(hs_hbm.at[src_v.at[w]], rows[b], gsem[b])

        def g_wait(w, b):
            pltpu.make_async_copy(hs_hbm.at[src_v.at[w]], rows[b],
                                  gsem[b]).wait()

        def s_start(w, b):
            return pltpu.async_copy(rows[b], acc_sh.at[dst_v.at[w]], ssem[b],
                                    add=True)

        def s_wait(w, b):
            pltpu.make_async_copy(rows[b], acc_sh.at[dst_v.at[w]],
                                  ssem[b]).wait()

        for p in range(nph):
            base = wid * ch + p * chp
            pltpu.sync_copy(src_hbm.at[pl.ds(base, chp)], src_v)
            pltpu.sync_copy(dst_hbm.at[pl.ds(base, chp)], dst_v)

            g_start(0, 0)
            g_start(1, 1)

            @pl.loop(0, chp, step=nslot)
            def _(j):
                for b in range(nslot):
                    w = j + b
                    g_wait(w, b)
                    cp = s_start(w, b)

                    @pl.when(w + 2 < chp)
                    def _():
                        g_start(w + 2, (b + 2) % nslot)

                    cp.wait()

        plsc.subcore_barrier()

        @pl.loop(0, rpt // ck)
        def _(j):
            r0 = sid * rpt + j * ck
            pltpu.sync_copy(acc_sh.at[pl.ds(r0, ck)], rows0)
            pltpu.sync_copy(rows0, out_hbm.at[pl.ds(cid * npad + r0, ck)])

    return k(hs, src3, dst3)


_HI = lax.Precision.HIGHEST


def _prep_body(d0_ref, d1_ref, x_ref, w_ref, b_ref, dis_ref, xs_ref, lg_ref):
    deg = d0_ref[:, 0:1] + d1_ref[:, 0:1] + 1.0
    dis = lax.rsqrt(deg)
    dis_ref[...] = dis
    xs_ref[...] = x_ref[...] * dis
    lg_ref[...] = jnp.dot(x_ref[...], w_ref[...],
                          preferred_element_type=jnp.float32,
                          precision=_HI) + b_ref[...]


def _tc_prep(degp, x_pad, w0, bout, npad, rblk):
    nb = npad // rblk
    return pl.pallas_call(
        _prep_body,
        grid=(nb,),
        in_specs=[
            pl.BlockSpec((rblk, _DEGW), lambda i: (i, 0)),
            pl.BlockSpec((rblk, _DEGW), lambda i, nb=nb: (i + nb, 0)),
            pl.BlockSpec((rblk, _D), lambda i: (i, 0)),
            pl.BlockSpec((_D, _C), lambda i: (0, 0)),
            pl.BlockSpec((1, _C), lambda i: (0, 0)),
        ],
        out_specs=[
            pl.BlockSpec((rblk, 1), lambda i: (i, 0)),
            pl.BlockSpec((rblk, _D), lambda i: (i, 0)),
            pl.BlockSpec((rblk, _C), lambda i: (i, 0)),
        ],
        out_shape=[
            jax.ShapeDtypeStruct((npad, 1), jnp.float32),
            jax.ShapeDtypeStruct((npad, _D), jnp.float32),
            jax.ShapeDtypeStruct((npad, _C), jnp.float32),
        ],
    )(degp, degp, x_pad, w0, bout)


def _layer_body(p0_ref, p1_ref, hs_ref, dis_ref, w_ref, b_ref, wo_ref,
                lgin_ref, hso_ref, lgo_ref):
    dis = dis_ref[...]
    agg = (p0_ref[...] + p1_ref[...] + hs_ref[...]) * dis
    h = jnp.maximum(
        jnp.dot(agg, w_ref[...], preferred_element_type=jnp.float32,
                precision=_HI) + b_ref[...], 0.0)
    hso_ref[...] = h * dis
    lgo_ref[...] = lgin_ref[...] + jnp.dot(
        h, wo_ref[...], preferred_element_type=jnp.float32, precision=_HI)


def _tc_layer(parts, hs, dis, w, b, wo, lgin, npad, rblk):
    nb = npad // rblk
    return pl.pallas_call(
        _layer_body,
        grid=(nb,),
        in_specs=[
            pl.BlockSpec((rblk, _D), lambda i: (i, 0)),
            pl.BlockSpec((rblk, _D), lambda i, nb=nb: (i + nb, 0)),
            pl.BlockSpec((rblk, _D), lambda i: (i, 0)),
            pl.BlockSpec((rblk, 1), lambda i: (i, 0)),
            pl.BlockSpec((_D, _D), lambda i: (0, 0)),
            pl.BlockSpec((1, _D), lambda i: (0, 0)),
            pl.BlockSpec((_D, _C), lambda i: (0, 0)),
            pl.BlockSpec((rblk, _C), lambda i: (i, 0)),
        ],
        out_specs=[
            pl.BlockSpec((rblk, _D), lambda i: (i, 0)),
            pl.BlockSpec((rblk, _C), lambda i: (i, 0)),
        ],
        out_shape=[
            jax.ShapeDtypeStruct((npad, _D), jnp.float32),
            jax.ShapeDtypeStruct((npad, _C), jnp.float32),
        ],
    )(parts, parts, hs, dis, w, b, wo, lgin)


def kernel(x, edge_index, W1, b1, W2, b2, W3, b3, Wout, bout):
    n, d = x.shape
    e = edge_index.shape[1]
    npad = ((n + 16 * _CK - 1) // (16 * _CK)) * (16 * _CK)
    ckw = 64
    gran = _NW * ckw * 8
    epw = ((e + gran - 1) // gran) * gran // _NW
    epad = epw * _NW

    pad = epad - e
    pad_idx = n + (jnp.arange(pad, dtype=jnp.int32) % (npad - n))
    src_p = jnp.concatenate([edge_index[0], pad_idx]).reshape(epad // ckw, ckw)
    dst_p = jnp.concatenate([edge_index[1], pad_idx]).reshape(epad // ckw, ckw)

    x_pad = jnp.pad(x, ((0, npad - n), (0, 0)))
    bout2 = bout.reshape(1, _C)

    rblk = 1024

    degp = _sc_degree(dst_p, npad)
    dis, xs, lg0 = _tc_prep(degp, x_pad, Wout[0:_D], bout2, npad, rblk)

    p1 = _sc_propagate(xs, src_p, dst_p)
    hs1, lg1 = _tc_layer(p1, xs, dis, W1, b1.reshape(1, _D),
                         Wout[_D:2 * _D], lg0, npad, rblk)

    p2 = _sc_propagate(hs1, src_p, dst_p)
    hs2, lg2 = _tc_layer(p2, hs1, dis, W2, b2.reshape(1, _D),
                         Wout[2 * _D:3 * _D], lg1, npad, rblk)

    p3 = _sc_propagate(hs2, src_p, dst_p)
    _, lg3 = _tc_layer(p3, hs2, dis, W3, b3.reshape(1, _D),
                       Wout[3 * _D:4 * _D], lg2, npad, rblk)

    return lg3[:n]


# speedup vs baseline: 23.7354x; 1.0124x over previous
import functools

import jax
import jax.numpy as jnp
from jax import lax
from jax.experimental import pallas as pl
from jax.experimental.pallas import tpu as pltpu
from jax.experimental.pallas import tpu_sc as plsc

_D = 128
_C = 40
_CK = 128
_NSUB = 16
_NCORE = 2
_NW = _NCORE * _NSUB
_LANES = 16
_DEGW = 16


def _sc_degree(dst3, npad):
    nwch, ck = dst3.shape
    ch = nwch // _NW
    rpt = npad // _NSUB
    mesh = plsc.VectorSubcoreMesh(core_axis_name="c", subcore_axis_name="s")

    @functools.partial(
        pl.kernel,
        out_type=jax.ShapeDtypeStruct((2 * npad, _DEGW), jnp.float32),
        mesh=mesh,
        scratch_types=[
            pltpu.VMEM((ch, ck), jnp.int32),
            pltpu.VMEM((ck, _DEGW), jnp.float32),
            pltpu.VMEM_SHARED((npad, _DEGW), jnp.float32),
            pltpu.SemaphoreType.DMA,
        ],
    )
    def k(dst_hbm, out_hbm, dst_v, buf_v, acc_sh, sem):
        cid = lax.axis_index("c")
        sid = lax.axis_index("s")
        wid = cid * _NSUB + sid

        @pl.loop(0, ck)
        def _(i):
            buf_v.at[i][...] = jnp.zeros((_DEGW,), jnp.float32)

        @pl.loop(0, rpt // ck)
        def _(j):
            pltpu.sync_copy(buf_v, acc_sh.at[pl.ds(sid * rpt + j * ck, ck)])

        @pl.loop(0, ck)
        def _(i):
            buf_v.at[i][...] = jnp.ones((_DEGW,), jnp.float32)

        pltpu.sync_copy(dst_hbm.at[pl.ds(wid * ch, ch)], dst_v)
        plsc.subcore_barrier()

        @pl.loop(0, ch)
        def _(j):
            pltpu.async_copy(buf_v, acc_sh.at[dst_v.at[j]], sem, add=True)

        @pl.loop(0, ch)
        def _(j):
            pltpu.make_async_copy(buf_v, acc_sh.at[dst_v.at[j]], sem).wait()

        plsc.subcore_barrier()

        @pl.loop(0, rpt // ck)
        def _(j):
            r0 = sid * rpt + j * ck
            pltpu.sync_copy(acc_sh.at[pl.ds(r0, ck)], buf_v)
            pltpu.sync_copy(buf_v, out_hbm.at[pl.ds(cid * npad + r0, ck)])

    return k(dst3)


def _sc_propagate(hs, src3, dst3):
    npad = hs.shape[0]
    nwch, ck = src3.shape
    ch = nwch // _NW
    rpt = npad // _NSUB
    mesh = plsc.VectorSubcoreMesh(core_axis_name="c", subcore_axis_name="s")

    nslot = 4
    nph = 4
    chp = ch // nph
    assert ch % (nph * nslot) == 0

    @functools.partial(
        pl.kernel,
        out_type=jax.ShapeDtypeStruct((2 * npad, _D), jnp.float32),
        mesh=mesh,
        scratch_types=[
            pltpu.VMEM((chp, ck), jnp.int32),
            pltpu.VMEM((chp, ck), jnp.int32),
            pltpu.VMEM_SHARED((npad, _D), jnp.float32),
        ] + [pltpu.VMEM((ck, _D), jnp.float32)] * nslot
          + [pltpu.SemaphoreType.DMA] * (2 * nslot),
    )
    def k(hs_hbm, src_hbm, dst_hbm, out_hbm, src_v, dst_v, acc_sh, *rest):
        rows = rest[:nslot]
        gsem = rest[nslot:2 * nslot]
        ssem = rest[2 * nslot:]
        rows0 = rows[0]
        cid = lax.axis_index("c")
        sid = lax.axis_index("s")
        wid = cid * _NSUB + sid

        @pl.loop(0, ck)
        def _(i):
            @pl.loop(0, _D // _LANES)
            def _(j):
                rows0.at[i, pl.ds(j * _LANES, _LANES)][...] = jnp.zeros(
                    (_LANES,), jnp.float32)

        @pl.loop(0, rpt // ck)
        def _(j):
            pltpu.async_copy(rows0,
                             acc_sh.at[pl.ds(sid * rpt + j * ck, ck)],
                             gsem[3])

        @pl.loop(0, rpt // ck)
        def _(j):
            pltpu.make_async_copy(rows0,
                                  acc_sh.at[pl.ds(sid * rpt + j * ck, ck)],
                                  gsem[3]).wait()

        plsc.subcore_barrier()

        def g_start(w, b):
            pltpu.async_copy(hs_hbm.at[src_v.at[w]], rows[b], gsem[b])

        def g_wait(w, b):
            pltpu.make_async_copy(hs_hbm.at[src_v.at[w]], rows[b],
                                  gsem[b]).wait()

        def s_start(w, b):
            return pltpu.async_copy(rows[b], acc_sh.at[dst_v.at[w]], ssem[b],
                                    add=True)

        for p in range(nph):
            base = wid * ch + p * chp
            pltpu.sync_copy(src_hbm.at[pl.ds(base, chp)], src_v)
            pltpu.sync_copy(dst_hbm.at[pl.ds(base, chp)], dst_v)

            g_start(0, 0)
            g_start(1, 1)

            @pl.loop(0, chp, step=nslot)
            def _(j):
                for b in range(nslot):
                    w = j + b
                    g_wait(w, b)
                    cp = s_start(w, b)

                    @pl.when(w + 2 < chp)
                    def _():
                        g_start(w + 2, (b + 2) % nslot)

                    cp.wait()

        plsc.subcore_barrier()

        @pl.loop(0, rpt // ck)
        def _(j):
            r0 = sid * rpt + j * ck
            pltpu.sync_copy(acc_sh.at[pl.ds(r0, ck)], rows0)
            pltpu.sync_copy(rows0, out_hbm.at[pl.ds(cid * npad + r0, ck)])

    return k(hs, src3, dst3)


_HI = lax.Precision.HIGHEST


def _prep_body(d0_ref, d1_ref, x_ref, w_ref, b_ref, dis_ref, xs_ref, lg_ref):
    deg = d0_ref[:, 0:1] + d1_ref[:, 0:1] + 1.0
    dis = lax.rsqrt(deg)
    dis_ref[...] = dis
    xs_ref[...] = x_ref[...] * dis
    lg_ref[...] = jnp.dot(x_ref[...], w_ref[...],
                          preferred_element_type=jnp.float32,
                          precision=_HI) + b_ref[...]


def _tc_prep(degp, x_pad, w0, bout, npad, rblk):
    nb = npad // rblk
    return pl.pallas_call(
        _prep_body,
        grid=(nb,),
        in_specs=[
            pl.BlockSpec((rblk, _DEGW), lambda i: (i, 0)),
            pl.BlockSpec((rblk, _DEGW), lambda i, nb=nb: (i + nb, 0)),
            pl.BlockSpec((rblk, _D), lambda i: (i, 0)),
            pl.BlockSpec((_D, _C), lambda i: (0, 0)),
            pl.BlockSpec((1, _C), lambda i: (0, 0)),
        ],
        out_specs=[
            pl.BlockSpec((rblk, 1), lambda i: (i, 0)),
            pl.BlockSpec((rblk, _D), lambda i: (i, 0)),
            pl.BlockSpec((rblk, _C), lambda i: (i, 0)),
        ],
        out_shape=[
            jax.ShapeDtypeStruct((npad, 1), jnp.float32),
            jax.ShapeDtypeStruct((npad, _D), jnp.float32),
            jax.ShapeDtypeStruct((npad, _C), jnp.float32),
        ],
    )(degp, degp, x_pad, w0, bout)


def _layer_body(p0_ref, p1_ref, hs_ref, dis_ref, w_ref, b_ref, wo_ref,
                lgin_ref, hso_ref, lgo_ref):
    dis = dis_ref[...]
    agg = (p0_ref[...] + p1_ref[...] + hs_ref[...]) * dis
    h = jnp.maximum(
        jnp.dot(agg, w_ref[...], preferred_element_type=jnp.float32,
                precision=_HI) + b_ref[...], 0.0)
    hso_ref[...] = h * dis
    lgo_ref[...] = lgin_ref[...] + jnp.dot(
        h, wo_ref[...], preferred_element_type=jnp.float32, precision=_HI)


def _tc_layer(parts, hs, dis, w, b, wo, lgin, npad, rblk):
    nb = npad // rblk
    return pl.pallas_call(
        _layer_body,
        grid=(nb,),
        in_specs=[
            pl.BlockSpec((rblk, _D), lambda i: (i, 0)),
            pl.BlockSpec((rblk, _D), lambda i, nb=nb: (i + nb, 0)),
            pl.BlockSpec((rblk, _D), lambda i: (i, 0)),
            pl.BlockSpec((rblk, 1), lambda i: (i, 0)),
            pl.BlockSpec((_D, _D), lambda i: (0, 0)),
            pl.BlockSpec((1, _D), lambda i: (0, 0)),
            pl.BlockSpec((_D, _C), lambda i: (0, 0)),
            pl.BlockSpec((rblk, _C), lambda i: (i, 0)),
        ],
        out_specs=[
            pl.BlockSpec((rblk, _D), lambda i: (i, 0)),
            pl.BlockSpec((rblk, _C), lambda i: (i, 0)),
        ],
        out_shape=[
            jax.ShapeDtypeStruct((npad, _D), jnp.float32),
            jax.ShapeDtypeStruct((npad, _C), jnp.float32),
        ],
    )(parts, parts, hs, dis, w, b, wo, lgin)


def kernel(x, edge_index, W1, b1, W2, b2, W3, b3, Wout, bout):
    n, d = x.shape
    e = edge_index.shape[1]
    npad = ((n + 16 * _CK - 1) // (16 * _CK)) * (16 * _CK)
    ckw = 64
    gran = _NW * ckw * 8
    epw = ((e + gran - 1) // gran) * gran // _NW
    epad = epw * _NW

    pad = epad - e
    pad_idx = n + (jnp.arange(pad, dtype=jnp.int32) % (npad - n))
    src_p = jnp.concatenate([edge_index[0], pad_idx]).reshape(epad // ckw, ckw)
    dst_p = jnp.concatenate([edge_index[1], pad_idx]).reshape(epad // ckw, ckw)

    x_pad = jnp.pad(x, ((0, npad - n), (0, 0)))
    bout2 = bout.reshape(1, _C)

    rblk = 2048

    degp = _sc_degree(dst_p, npad)
    dis, xs, lg0 = _tc_prep(degp, x_pad, Wout[0:_D], bout2, npad, rblk)

    p1 = _sc_propagate(xs, src_p, dst_p)
    hs1, lg1 = _tc_layer(p1, xs, dis, W1, b1.reshape(1, _D),
                         Wout[_D:2 * _D], lg0, npad, rblk)

    p2 = _sc_propagate(hs1, src_p, dst_p)
    hs2, lg2 = _tc_layer(p2, hs1, dis, W2, b2.reshape(1, _D),
                         Wout[2 * _D:3 * _D], lg1, npad, rblk)

    p3 = _sc_propagate(hs2, src_p, dst_p)
    _, lg3 = _tc_layer(p3, hs2, dis, W3, b3.reshape(1, _D),
                       Wout[3 * _D:4 * _D], lg2, npad, rblk)

    return lg3[:n]


# speedup vs baseline: 23.7477x; 1.0005x over previous
import functools

import jax
import jax.numpy as jnp
from jax import lax
from jax.experimental import pallas as pl
from jax.experimental.pallas import tpu as pltpu
from jax.experimental.pallas import tpu_sc as plsc

_D = 128
_C = 40
_CK = 128
_NSUB = 16
_NCORE = 2
_NW = _NCORE * _NSUB
_LANES = 16
_DEGW = 16


def _sc_degree(dst3, npad):
    nwch, ck = dst3.shape
    ch = nwch // _NW
    rpt = npad // _NSUB
    mesh = plsc.VectorSubcoreMesh(core_axis_name="c", subcore_axis_name="s")

    @functools.partial(
        pl.kernel,
        out_type=jax.ShapeDtypeStruct((2 * npad, _DEGW), jnp.float32),
        mesh=mesh,
        scratch_types=[
            pltpu.VMEM((ch, ck), jnp.int32),
            pltpu.VMEM((ck, _DEGW), jnp.float32),
            pltpu.VMEM_SHARED((npad, _DEGW), jnp.float32),
            pltpu.SemaphoreType.DMA,
        ],
    )
    def k(dst_hbm, out_hbm, dst_v, buf_v, acc_sh, sem):
        cid = lax.axis_index("c")
        sid = lax.axis_index("s")
        wid = cid * _NSUB + sid

        @pl.loop(0, ck)
        def _(i):
            buf_v.at[i][...] = jnp.zeros((_DEGW,), jnp.float32)

        @pl.loop(0, rpt // ck)
        def _(j):
            pltpu.sync_copy(buf_v, acc_sh.at[pl.ds(sid * rpt + j * ck, ck)])

        @pl.loop(0, ck)
        def _(i):
            buf_v.at[i][...] = jnp.ones((_DEGW,), jnp.float32)

        pltpu.sync_copy(dst_hbm.at[pl.ds(wid * ch, ch)], dst_v)
        plsc.subcore_barrier()

        @pl.loop(0, ch)
        def _(j):
            pltpu.async_copy(buf_v, acc_sh.at[dst_v.at[j]], sem, add=True)

        @pl.loop(0, ch)
        def _(j):
            pltpu.make_async_copy(buf_v, acc_sh.at[dst_v.at[j]], sem).wait()

        plsc.subcore_barrier()

        @pl.loop(0, rpt // ck)
        def _(j):
            r0 = sid * rpt + j * ck
            pltpu.sync_copy(acc_sh.at[pl.ds(r0, ck)], buf_v)
            pltpu.sync_copy(buf_v, out_hbm.at[pl.ds(cid * npad + r0, ck)])

    return k(dst3)


def _sc_propagate(hs, src3, dst3):
    npad = hs.shape[0]
    nwch, ck = src3.shape
    ch = nwch // _NW
    rpt = npad // _NSUB
    mesh = plsc.VectorSubcoreMesh(core_axis_name="c", subcore_axis_name="s")

    nslot = 4
    nph = 4
    chp = ch // nph
    assert ch % (nph * nslot) == 0

    @functools.partial(
        pl.kernel,
        out_type=jax.ShapeDtypeStruct((2 * npad, _D), jnp.float32),
        mesh=mesh,
        scratch_types=[
            pltpu.VMEM((chp, ck), jnp.int32),
            pltpu.VMEM((chp, ck), jnp.int32),
            pltpu.VMEM_SHARED((npad, _D), jnp.float32),
        ] + [pltpu.VMEM((ck, _D), jnp.float32)] * nslot
          + [pltpu.SemaphoreType.DMA] * (2 * nslot),
    )
    def k(hs_hbm, src_hbm, dst_hbm, out_hbm, src_v, dst_v, acc_sh, *rest):
        rows = rest[:nslot]
        gsem = rest[nslot:2 * nslot]
        ssem = rest[2 * nslot:]
        rows0 = rows[0]
        cid = lax.axis_index("c")
        sid = lax.axis_index("s")
        wid = cid * _NSUB + sid

        @pl.loop(0, ck)
        def _(i):
            @pl.loop(0, _D // _LANES)
            def _(j):
                rows0.at[i, pl.ds(j * _LANES, _LANES)][...] = jnp.zeros(
                    (_LANES,), jnp.float32)

        @pl.loop(0, rpt // ck)
        def _(j):
            pltpu.sync_copy(rows0,
                            acc_sh.at[pl.ds(sid * rpt + j * ck, ck)])

        plsc.subcore_barrier()

        def g_start(w, b):
            pltpu.async_copy(hs_hbm.at[src_v.at[w]], rows[b], gsem[b])

        def g_wait(w, b):
            pltpu.make_async_copy(hs_hbm.at[src_v.at[w]], rows[b],
                                  gsem[b]).wait()

        def s_start(w, b):
            return pltpu.async_copy(rows[b], acc_sh.at[dst_v.at[w]], ssem[b],
                                    add=True)

        for p in range(nph):
            base = wid * ch + p * chp
            pltpu.sync_copy(src_hbm.at[pl.ds(base, chp)], src_v)
            pltpu.sync_copy(dst_hbm.at[pl.ds(base, chp)], dst_v)

            g_start(0, 0)
            g_start(1, 1)

            @pl.loop(0, chp, step=nslot)
            def _(j):
                for b in range(nslot):
                    w = j + b
                    g_wait(w, b)
                    cp = s_start(w, b)

                    @pl.when(w + 2 < chp)
                    def _():
                        g_start(w + 2, (b + 2) % nslot)

                    cp.wait()

        plsc.subcore_barrier()

        @pl.loop(0, rpt // ck)
        def _(j):
            r0 = sid * rpt + j * ck
            pltpu.sync_copy(acc_sh.at[pl.ds(r0, ck)], rows0)
            pltpu.sync_copy(rows0, out_hbm.at[pl.ds(cid * npad + r0, ck)])

    return k(hs, src3, dst3)


_HI = lax.Precision.HIGHEST


def _prep_body(d0_ref, d1_ref, x_ref, w_ref, b_ref, dis_ref, xs_ref, lg_ref):
    deg = d0_ref[:, 0:1] + d1_ref[:, 0:1] + 1.0
    dis = lax.rsqrt(deg)
    dis_ref[...] = dis
    xs_ref[...] = x_ref[...] * dis
    lg_ref[...] = jnp.dot(x_ref[...], w_ref[...],
                          preferred_element_type=jnp.float32,
                          precision=_HI) + b_ref[...]


def _tc_prep(degp, x_pad, w0, bout, npad, rblk):
    nb = npad // rblk
    return pl.pallas_call(
        _prep_body,
        grid=(nb,),
        in_specs=[
            pl.BlockSpec((rblk, _DEGW), lambda i: (i, 0)),
            pl.BlockSpec((rblk, _DEGW), lambda i, nb=nb: (i + nb, 0)),
            pl.BlockSpec((rblk, _D), lambda i: (i, 0)),
            pl.BlockSpec((_D, _C), lambda i: (0, 0)),
            pl.BlockSpec((1, _C), lambda i: (0, 0)),
        ],
        out_specs=[
            pl.BlockSpec((rblk, 1), lambda i: (i, 0)),
            pl.BlockSpec((rblk, _D), lambda i: (i, 0)),
            pl.BlockSpec((rblk, _C), lambda i: (i, 0)),
        ],
        out_shape=[
            jax.ShapeDtypeStruct((npad, 1), jnp.float32),
            jax.ShapeDtypeStruct((npad, _D), jnp.float32),
            jax.ShapeDtypeStruct((npad, _C), jnp.float32),
        ],
    )(degp, degp, x_pad, w0, bout)


def _layer_body(p0_ref, p1_ref, hs_ref, dis_ref, w_ref, b_ref, wo_ref,
                lgin_ref, hso_ref, lgo_ref):
    dis = dis_ref[...]
    agg = (p0_ref[...] + p1_ref[...] + hs_ref[...]) * dis
    h = jnp.maximum(
        jnp.dot(agg, w_ref[...], preferred_element_type=jnp.float32,
                precision=_HI) + b_ref[...], 0.0)
    hso_ref[...] = h * dis
    lgo_ref[...] = lgin_ref[...] + jnp.dot(
        h, wo_ref[...], preferred_element_type=jnp.float32, precision=_HI)


def _tc_layer(parts, hs, dis, w, b, wo, lgin, npad, rblk):
    nb = npad // rblk
    return pl.pallas_call(
        _layer_body,
        grid=(nb,),
        in_specs=[
            pl.BlockSpec((rblk, _D), lambda i: (i, 0)),
            pl.BlockSpec((rblk, _D), lambda i, nb=nb: (i + nb, 0)),
            pl.BlockSpec((rblk, _D), lambda i: (i, 0)),
            pl.BlockSpec((rblk, 1), lambda i: (i, 0)),
            pl.BlockSpec((_D, _D), lambda i: (0, 0)),
            pl.BlockSpec((1, _D), lambda i: (0, 0)),
            pl.BlockSpec((_D, _C), lambda i: (0, 0)),
            pl.BlockSpec((rblk, _C), lambda i: (i, 0)),
        ],
        out_specs=[
            pl.BlockSpec((rblk, _D), lambda i: (i, 0)),
            pl.BlockSpec((rblk, _C), lambda i: (i, 0)),
        ],
        out_shape=[
            jax.ShapeDtypeStruct((npad, _D), jnp.float32),
            jax.ShapeDtypeStruct((npad, _C), jnp.float32),
        ],
    )(parts, parts, hs, dis, w, b, wo, lgin)


def kernel(x, edge_index, W1, b1, W2, b2, W3, b3, Wout, bout):
    n, d = x.shape
    e = edge_index.shape[1]
    npad = ((n + 16 * _CK - 1) // (16 * _CK)) * (16 * _CK)
    ckw = 64
    gran = _NW * ckw * 8
    epw = ((e + gran - 1) // gran) * gran // _NW
    epad = epw * _NW

    pad = epad - e
    pad_idx = n + (jnp.arange(pad, dtype=jnp.int32) % (npad - n))
    src_cat = jnp.concatenate([edge_index[0], pad_idx])
    dst_cat = jnp.concatenate([edge_index[1], pad_idx])
    src_p = src_cat.reshape(epad // ckw, ckw)
    dst_p = dst_cat.reshape(epad // ckw, ckw)
    dst_q = dst_cat.reshape(epad // _CK, _CK)

    x_pad = jnp.pad(x, ((0, npad - n), (0, 0)))
    bout2 = bout.reshape(1, _C)

    rblk = 2048

    degp = _sc_degree(dst_q, npad)
    dis, xs, lg0 = _tc_prep(degp, x_pad, Wout[0:_D], bout2, npad, rblk)

    p1 = _sc_propagate(xs, src_p, dst_p)
    hs1, lg1 = _tc_layer(p1, xs, dis, W1, b1.reshape(1, _D),
                         Wout[_D:2 * _D], lg0, npad, rblk)

    p2 = _sc_propagate(hs1, src_p, dst_p)
    hs2, lg2 = _tc_layer(p2, hs1, dis, W2, b2.reshape(1, _D),
                         Wout[2 * _D:3 * _D], lg1, npad, rblk)

    p3 = _sc_propagate(hs2, src_p, dst_p)
    _, lg3 = _tc_layer(p3, hs2, dis, W3, b3.reshape(1, _D),
                       Wout[3 * _D:4 * _D], lg2, npad, rblk)

    return lg3[:n]


# speedup vs baseline: 23.7737x; 1.0011x over previous
import functools

import jax
import jax.numpy as jnp
from jax import lax
from jax.experimental import pallas as pl
from jax.experimental.pallas import tpu as pltpu
from jax.experimental.pallas import tpu_sc as plsc

_D = 128
_C = 40
_CK = 128
_NSUB = 16
_NCORE = 2
_NW = _NCORE * _NSUB
_LANES = 16
_DEGW = 16


def _sc_degree(dst3, npad):
    nwch, ck = dst3.shape
    ch = nwch // _NW
    rpt = npad // _NSUB
    mesh = plsc.VectorSubcoreMesh(core_axis_name="c", subcore_axis_name="s")

    @functools.partial(
        pl.kernel,
        out_type=jax.ShapeDtypeStruct((2 * npad, _DEGW), jnp.float32),
        mesh=mesh,
        scratch_types=[
            pltpu.VMEM((ch, ck), jnp.int32),
            pltpu.VMEM((ck, _DEGW), jnp.float32),
            pltpu.VMEM_SHARED((npad, _DEGW), jnp.float32),
            pltpu.SemaphoreType.DMA,
        ],
    )
    def k(dst_hbm, out_hbm, dst_v, buf_v, acc_sh, sem):
        cid = lax.axis_index("c")
        sid = lax.axis_index("s")
        wid = cid * _NSUB + sid

        @pl.loop(0, ck)
        def _(i):
            buf_v.at[i][...] = jnp.zeros((_DEGW,), jnp.float32)

        @pl.loop(0, rpt // ck)
        def _(j):
            pltpu.sync_copy(buf_v, acc_sh.at[pl.ds(sid * rpt + j * ck, ck)])

        @pl.loop(0, ck)
        def _(i):
            buf_v.at[i][...] = jnp.ones((_DEGW,), jnp.float32)

        pltpu.sync_copy(dst_hbm.at[pl.ds(wid * ch, ch)], dst_v)
        plsc.subcore_barrier()

        @pl.loop(0, ch)
        def _(j):
            pltpu.async_copy(buf_v, acc_sh.at[dst_v.at[j]], sem, add=True)

        @pl.loop(0, ch)
        def _(j):
            pltpu.make_async_copy(buf_v, acc_sh.at[dst_v.at[j]], sem).wait()

        plsc.subcore_barrier()

        @pl.loop(0, rpt // ck)
        def _(j):
            r0 = sid * rpt + j * ck
            pltpu.sync_copy(acc_sh.at[pl.ds(r0, ck)], buf_v)
            pltpu.sync_copy(buf_v, out_hbm.at[pl.ds(cid * npad + r0, ck)])

    return k(dst3)


def _sc_propagate(hs, src3, dst3):
    npad = hs.shape[0]
    nwch, ck = src3.shape
    ch = nwch // _NW
    rpt = npad // _NSUB
    mesh = plsc.VectorSubcoreMesh(core_axis_name="c", subcore_axis_name="s")

    nslot = 4
    nph = 4
    chp = ch // nph
    assert ch % (nph * nslot) == 0

    @functools.partial(
        pl.kernel,
        out_type=jax.ShapeDtypeStruct((2 * npad, _D), jnp.float32),
        mesh=mesh,
        scratch_types=[
            pltpu.VMEM((chp, ck), jnp.int32),
            pltpu.VMEM((chp, ck), jnp.int32),
            pltpu.VMEM_SHARED((npad, _D), jnp.float32),
        ] + [pltpu.VMEM((ck, _D), jnp.float32)] * nslot
          + [pltpu.SemaphoreType.DMA] * (2 * nslot),
    )
    def k(hs_hbm, src_hbm, dst_hbm, out_hbm, src_v, dst_v, acc_sh, *rest):
        rows = rest[:nslot]
        gsem = rest[nslot:2 * nslot]
        ssem = rest[2 * nslot:]
        rows0 = rows[0]
        cid = lax.axis_index("c")
        sid = lax.axis_index("s")
        wid = cid * _NSUB + sid

        @pl.loop(0, ck)
        def _(i):
            @pl.loop(0, _D // _LANES)
            def _(j):
                rows0.at[i, pl.ds(j * _LANES, _LANES)][...] = jnp.zeros(
                    (_LANES,), jnp.float32)

        @pl.loop(0, rpt // ck)
        def _(j):
            pltpu.sync_copy(rows0,
                            acc_sh.at[pl.ds(sid * rpt + j * ck, ck)])

        plsc.subcore_barrier()

        def g_start(w, b):
            pltpu.async_copy(hs_hbm.at[src_v.at[w]], rows[b], gsem[b])

        def g_wait(w, b):
            pltpu.make_async_copy(hs_hbm.at[src_v.at[w]], rows[b],
                                  gsem[b]).wait()

        def s_start(w, b):
            return pltpu.async_copy(rows[b], acc_sh.at[dst_v.at[w]], ssem[b],
                                    add=True)

        for p in range(nph):
            base = wid * ch + p * chp
            pltpu.sync_copy(src_hbm.at[pl.ds(base, chp)], src_v)
            pltpu.sync_copy(dst_hbm.at[pl.ds(base, chp)], dst_v)

            for b in range(nslot):
                g_start(b, b)

            @pl.loop(0, chp, step=nslot)
            def _(j):
                cps = [None] * nslot
                for b in range(nslot):
                    w = j + b
                    g_wait(w, b)
                    cps[b] = s_start(w, b)
                    if b >= 1:
                        cps[b - 1].wait()
                        wref = w - 1 + nslot

                        @pl.when(wref < chp)
                        def _(wref=wref, bb=b - 1):
                            g_start(wref, bb)

                cps[nslot - 1].wait()
                wlast = j + nslot - 1 + nslot

                @pl.when(wlast < chp)
                def _():
                    g_start(wlast, nslot - 1)

        plsc.subcore_barrier()

        @pl.loop(0, rpt // ck)
        def _(j):
            r0 = sid * rpt + j * ck
            pltpu.sync_copy(acc_sh.at[pl.ds(r0, ck)], rows0)
            pltpu.sync_copy(rows0, out_hbm.at[pl.ds(cid * npad + r0, ck)])

    return k(hs, src3, dst3)


_HI = lax.Precision.HIGHEST


def _prep_body(d0_ref, d1_ref, x_ref, w_ref, b_ref, dis_ref, xs_ref, lg_ref):
    deg = d0_ref[:, 0:1] + d1_ref[:, 0:1] + 1.0
    dis = lax.rsqrt(deg)
    dis_ref[...] = dis
    xs_ref[...] = x_ref[...] * dis
    lg_ref[...] = jnp.dot(x_ref[...], w_ref[...],
                          preferred_element_type=jnp.float32,
                          precision=_HI) + b_ref[...]


def _tc_prep(degp, x_pad, w0, bout, npad, rblk):
    nb = npad // rblk
    return pl.pallas_call(
        _prep_body,
        grid=(nb,),
        in_specs=[
            pl.BlockSpec((rblk, _DEGW), lambda i: (i, 0)),
            pl.BlockSpec((rblk, _DEGW), lambda i, nb=nb: (i + nb, 0)),
            pl.BlockSpec((rblk, _D), lambda i: (i, 0)),
            pl.BlockSpec((_D, _C), lambda i: (0, 0)),
            pl.BlockSpec((1, _C), lambda i: (0, 0)),
        ],
        out_specs=[
            pl.BlockSpec((rblk, 1), lambda i: (i, 0)),
            pl.BlockSpec((rblk, _D), lambda i: (i, 0)),
            pl.BlockSpec((rblk, _C), lambda i: (i, 0)),
        ],
        out_shape=[
            jax.ShapeDtypeStruct((npad, 1), jnp.float32),
            jax.ShapeDtypeStruct((npad, _D), jnp.float32),
            jax.ShapeDtypeStruct((npad, _C), jnp.float32),
        ],
    )(degp, degp, x_pad, w0, bout)


def _layer_body(p0_ref, p1_ref, hs_ref, dis_ref, w_ref, b_ref, wo_ref,
                lgin_ref, hso_ref, lgo_ref):
    dis = dis_ref[...]
    agg = (p0_ref[...] + p1_ref[...] + hs_ref[...]) * dis
    h = jnp.maximum(
        jnp.dot(agg, w_ref[...], preferred_element_type=jnp.float32,
                precision=_HI) + b_ref[...], 0.0)
    hso_ref[...] = h * dis
    lgo_ref[...] = lgin_ref[...] + jnp.dot(
        h, wo_ref[...], preferred_element_type=jnp.float32, precision=_HI)


def _tc_layer(parts, hs, dis, w, b, wo, lgin, npad, rblk):
    nb = npad // rblk
    return pl.pallas_call(
        _layer_body,
        grid=(nb,),
        in_specs=[
            pl.BlockSpec((rblk, _D), lambda i: (i, 0)),
            pl.BlockSpec((rblk, _D), lambda i, nb=nb: (i + nb, 0)),
            pl.BlockSpec((rblk, _D), lambda i: (i, 0)),
            pl.BlockSpec((rblk, 1), lambda i: (i, 0)),
            pl.BlockSpec((_D, _D), lambda i: (0, 0)),
            pl.BlockSpec((1, _D), lambda i: (0, 0)),
            pl.BlockSpec((_D, _C), lambda i: (0, 0)),
            pl.BlockSpec((rblk, _C), lambda i: (i, 0)),
        ],
        out_specs=[
            pl.BlockSpec((rblk, _D), lambda i: (i, 0)),
            pl.BlockSpec((rblk, _C), lambda i: (i, 0)),
        ],
        out_shape=[
            jax.ShapeDtypeStruct((npad, _D), jnp.float32),
            jax.ShapeDtypeStruct((npad, _C), jnp.float32),
        ],
    )(parts, parts, hs, dis, w, b, wo, lgin)


def kernel(x, edge_index, W1, b1, W2, b2, W3, b3, Wout, bout):
    n, d = x.shape
    e = edge_index.shape[1]
    npad = ((n + 16 * _CK - 1) // (16 * _CK)) * (16 * _CK)
    ckw = 64
    gran = _NW * ckw * 8
    epw = ((e + gran - 1) // gran) * gran // _NW
    epad = epw * _NW

    pad = epad - e
    pad_idx = n + (jnp.arange(pad, dtype=jnp.int32) % (npad - n))
    src_cat = jnp.concatenate([edge_index[0], pad_idx])
    dst_cat = jnp.concatenate([edge_index[1], pad_idx])
    src_p = src_cat.reshape(epad // ckw, ckw)
    dst_p = dst_cat.reshape(epad // ckw, ckw)
    dst_q = dst_cat.reshape(epad // _CK, _CK)

    x_pad = jnp.pad(x, ((0, npad - n), (0, 0)))
    bout2 = bout.reshape(1, _C)

    rblk = 2048

    degp = _sc_degree(dst_q, npad)
    dis, xs, lg0 = _tc_prep(degp, x_pad, Wout[0:_D], bout2, npad, rblk)

    p1 = _sc_propagate(xs, src_p, dst_p)
    hs1, lg1 = _tc_layer(p1, xs, dis, W1, b1.reshape(1, _D),
                         Wout[_D:2 * _D], lg0, npad, rblk)

    p2 = _sc_propagate(hs1, src_p, dst_p)
    hs2, lg2 = _tc_layer(p2, hs1, dis, W2, b2.reshape(1, _D),
                         Wout[2 * _D:3 * _D], lg1, npad, rblk)

    p3 = _sc_propagate(hs2, src_p, dst_p)
    _, lg3 = _tc_layer(p3, hs2, dis, W3, b3.reshape(1, _D),
                       Wout[3 * _D:4 * _D], lg2, npad, rblk)

    return lg3[:n]
